# baseline (speedup 1.0000x reference)
import jax
import jax.numpy as jnp
from jax import lax
from jax.experimental import pallas as pl
from jax.experimental.pallas import tpu as pltpu
from jax.experimental.pallas import tpu_sc as plsc
from jax.sharding import PartitionSpec as P

N = 100000
C = 128
NC, NS = 2, 16
NW = NC * NS
CH = 224
G = CH // 16

NDEV = jax.device_count()
if N % NDEV or (N // NDEV) % 16:
    NDEV = 1
NLOC = N // NDEV
NCHUNK = -(-NLOC // (NW * CH))
R = NCHUNK * CH
RL = NLOC - (NW - 1) * R
FL = RL // CH
TL = RL - FL * CH
GL = TL // 16
IDXA = NW * R - NLOC + R
assert RL > 0 and RL % 16 == 0 and TL % 16 == 0
assert FL >= 2 and FL % 2 == 0 and (FL - 2) % 2 == 0 and GL > 0


def _onehot_body(idx_hbm, out_hbm, idx_v, buf0, buf1, sem0, sem1):
    bufs = (buf0, buf1)
    sems = (sem0, sem1)
    wid = lax.axis_index("s") * NC + lax.axis_index("c")
    islast = wid == NW - 1
    base = wid * R
    abase = pl.multiple_of(jnp.minimum(base, NLOC - R), 8)
    off = base - abase

    pltpu.sync_copy(idx_hbm.at[pl.ds(abase, R)], idx_v.at[pl.ds(0, R)])

    rows0 = lax.iota(jnp.int32, 16) * C
    ones = jnp.ones((16,), jnp.int32)
    zeros = jnp.zeros((16,), jnp.int32)

    def zbody(i, carry):
        buf0[pl.ds(i * 16, 16)] = zeros
        buf1[pl.ds(i * 16, 16)] = zeros
        return carry
    lax.fori_loop(0, CH * C // 16, zbody, 0, unroll=8)

    def drain(b, nrows=CH):
        pltpu.make_async_copy(
            bufs[b].at[pl.ds(0, nrows * C)],
            out_hbm.at[pl.ds(0, nrows * C)], sems[b]).wait()

    def clear_chunk(k, b):
        for g in range(G):
            old = idx_v[pl.ds(off + k * CH + g * 16, 16)]
            plsc.store_scatter(bufs[b], [rows0 + g * 16 * C + old], zeros)

    def ones_chunk(k, b, ngroups=G):
        for g in range(ngroups):
            cols = idx_v[pl.ds(off + k * CH + g * 16, 16)]
            plsc.store_scatter(bufs[b], [rows0 + g * 16 * C + cols], ones)

    def dma_out(k, b, nrows=CH):
        pltpu.async_copy(
            bufs[b].at[pl.ds(0, nrows * C)],
            out_hbm.at[pl.ds((base + k * CH) * C, nrows * C)],
            sems[b])

    for k in (0, 1):
        ones_chunk(k, k)
        dma_out(k, k)

    def loop_body(j, carry):
        k = 2 + 2 * j
        for b in range(2):
            drain(b)
            clear_chunk(k + b - 2, b)
            ones_chunk(k + b, b)
            dma_out(k + b, b)
        return carry
    lax.fori_loop(0, (FL - 2) // 2, loop_body, 0)

    for k in range(FL, NCHUNK):
        b = k % 2
        drain(b)
        clear_chunk(k - 2, b)

        @pl.when(jnp.logical_not(islast))
        def _():
            ones_chunk(k, b)
            dma_out(k, b)

        if k == FL:
            @pl.when(islast)
            def _():
                ones_chunk(k, b, ngroups=GL)
                dma_out(k, b, nrows=TL)

    for kk in (NCHUNK - 2, NCHUNK - 1):
        b = kk % 2
        if kk < FL:
            drain(b)
        else:
            @pl.when(jnp.logical_not(islast))
            def _():
                drain(b)

            if kk == FL:
                @pl.when(islast)
                def _():
                    drain(b, nrows=TL)


_onehot_sc = pl.kernel(
    _onehot_body,
    out_type=jax.ShapeDtypeStruct((NLOC * C,), jnp.int32),
    mesh=plsc.VectorSubcoreMesh(core_axis_name="c", subcore_axis_name="s"),
    compiler_params=pltpu.CompilerParams(needs_layout_passes=False),
    scratch_types=[
        pltpu.VMEM((IDXA,), jnp.int32),
        pltpu.VMEM((CH * C,), jnp.int32),
        pltpu.VMEM((CH * C,), jnp.int32),
        pltpu.SemaphoreType.DMA,
        pltpu.SemaphoreType.DMA,
    ],
)

_mesh = jax.make_mesh((NDEV,), ("d",),
                      axis_types=(jax.sharding.AxisType.Auto,))


def kernel(x, weight):
    del x
    out = jax.shard_map(
        _onehot_sc, mesh=_mesh, in_specs=P("d"), out_specs=P("d"))(weight)
    return out.reshape(N, C)


# speedup vs baseline: 1.0881x; 1.0881x over previous
import jax
import jax.numpy as jnp
from jax import lax
from jax.experimental import pallas as pl
from jax.experimental.pallas import tpu as pltpu
from jax.experimental.pallas import tpu_sc as plsc
from jax.sharding import PartitionSpec as P

N = 100000
C = 128
NC, NS = 2, 16
NW = NC * NS
CH = 224
G = CH // 16

NDEV = jax.device_count()
if N % NDEV or (N // NDEV) % 16:
    NDEV = 1
NLOC = N // NDEV
NCHUNK = -(-NLOC // (NW * CH))
R = NCHUNK * CH
RL = NLOC - (NW - 1) * R
FL = RL // CH
TL = RL - FL * CH
GL = TL // 16
IDXA = NW * R - NLOC + R
assert RL > 0 and RL % 16 == 0 and TL % 16 == 0
assert FL >= 2 and FL % 2 == 0 and (FL - 2) % 2 == 0 and GL > 0


def _onehot_body(idx_hbm, out_hbm, idx_v, buf0, buf1, sem0, sem1):
    bufs = (buf0, buf1)
    sems = (sem0, sem1)
    wid = lax.axis_index("s") * NC + lax.axis_index("c")
    islast = wid == NW - 1
    base = wid * R
    abase = pl.multiple_of(jnp.minimum(base, NLOC - R), 8)
    off = base - abase

    pltpu.sync_copy(idx_hbm.at[pl.ds(abase, R)], idx_v.at[pl.ds(0, R)])

    rows0 = lax.iota(jnp.int32, 16) * C
    ones = jnp.ones((16,), jnp.int32)
    zeros = jnp.zeros((16,), jnp.int32)

    def zbody(i, carry):
        buf0[pl.ds(i * 16, 16)] = zeros
        buf1[pl.ds(i * 16, 16)] = zeros
        return carry
    lax.fori_loop(0, CH * C // 16, zbody, 0, unroll=8)

    def drain(b, nrows=CH):
        pltpu.make_async_copy(
            bufs[b].at[pl.ds(0, nrows * C)],
            out_hbm.at[pl.ds(0, nrows * C)], sems[b]).wait()

    def clear_chunk(k, b):
        for g in range(G):
            old = idx_v[pl.ds(off + k * CH + g * 16, 16)]
            plsc.store_scatter(bufs[b], [rows0 + g * 16 * C + old], zeros)

    def ones_chunk(k, b, ngroups=G):
        for g in range(ngroups):
            cols = idx_v[pl.ds(off + k * CH + g * 16, 16)]
            plsc.store_scatter(bufs[b], [rows0 + g * 16 * C + cols], ones)

    def dma_out(k, b, nrows=CH):
        pltpu.async_copy(
            bufs[b].at[pl.ds(0, nrows * C)],
            out_hbm.at[pl.ds((base + k * CH) * C, nrows * C)],
            sems[b])

    for k in (0, 1):
        ones_chunk(k, k)
        dma_out(k, k)

    def loop_body(j, carry):
        k = 2 + 2 * j
        for b in range(2):
            drain(b)
            clear_chunk(k + b - 2, b)
            ones_chunk(k + b, b)
            dma_out(k + b, b)
        return carry
    lax.fori_loop(0, (FL - 2) // 2, loop_body, 0)

    for k in range(FL, NCHUNK):
        b = k % 2
        drain(b)
        clear_chunk(k - 2, b)

        @pl.when(jnp.logical_not(islast))
        def _():
            ones_chunk(k, b)
            dma_out(k, b)

        if k == FL:
            @pl.when(islast)
            def _():
                ones_chunk(k, b, ngroups=GL)
                dma_out(k, b, nrows=TL)

    for kk in (NCHUNK - 2, NCHUNK - 1):
        b = kk % 2
        if kk < FL:
            drain(b)
        else:
            @pl.when(jnp.logical_not(islast))
            def _():
                drain(b)

            if kk == FL:
                @pl.when(islast)
                def _():
                    drain(b, nrows=TL)


_onehot_sc = pl.kernel(
    _onehot_body,
    out_type=jax.ShapeDtypeStruct((NLOC * C,), jnp.int32),
    mesh=plsc.VectorSubcoreMesh(core_axis_name="c", subcore_axis_name="s"),
    compiler_params=pltpu.CompilerParams(needs_layout_passes=False),
    scratch_types=[
        pltpu.VMEM((IDXA,), jnp.int32),
        pltpu.VMEM((CH * C,), jnp.int32),
        pltpu.VMEM((CH * C,), jnp.int32),
        pltpu.SemaphoreType.DMA,
        pltpu.SemaphoreType.DMA,
    ],
)

_mesh = jax.make_mesh((NDEV,), ("d",),
                      axis_types=(jax.sharding.AxisType.Auto,))


def kernel(x, weight):
    del x
    out = jax.shard_map(
        _onehot_sc, mesh=_mesh, in_specs=P("d"), out_specs=P("d"))(weight)
    return jax.lax.with_sharding_constraint(
        out.reshape(N, C), jax.NamedSharding(_mesh, P("d", None)))


# speedup vs baseline: 3.5174x; 3.2326x over previous
import jax
import jax.numpy as jnp
from jax import lax
from jax.experimental import pallas as pl
from jax.experimental.pallas import tpu as pltpu
from jax.experimental.pallas import tpu_sc as plsc
from jax.sharding import PartitionSpec as P

N = 100000
C = 128
NC, NS = 2, 16
NW = NC * NS
CH = 224
G = CH // 16

NDEV = jax.device_count()
if N % NDEV or (N // NDEV) % 16:
    NDEV = 1
NLOC = N // NDEV
NCHUNK = -(-NLOC // (NW * CH))
R = NCHUNK * CH
RL = NLOC - (NW - 1) * R
FL = RL // CH
TL = RL - FL * CH
GL = TL // 16
IDXA = NW * R - NLOC + R
assert RL > 0 and RL % 16 == 0 and TL % 16 == 0
assert FL >= 2 and FL % 2 == 0 and (FL - 2) % 2 == 0 and GL > 0


def _onehot_body(idx_hbm, out_hbm, idx_v, buf0, buf1, sem0, sem1):
    bufs = (buf0, buf1)
    sems = (sem0, sem1)
    wid = lax.axis_index("s") * NC + lax.axis_index("c")
    islast = wid == NW - 1
    base = wid * R
    abase = pl.multiple_of(jnp.minimum(base, NLOC - R), 8)
    off = base - abase

    pltpu.sync_copy(idx_hbm.at[pl.ds(abase, R)], idx_v.at[pl.ds(0, R)])

    rows0 = lax.iota(jnp.int32, 16) * C
    ones = jnp.ones((16,), jnp.int32)
    zeros = jnp.zeros((16,), jnp.int32)

    def zbody(i, carry):
        buf0[pl.ds(i * 16, 16)] = zeros
        buf1[pl.ds(i * 16, 16)] = zeros
        return carry
    lax.fori_loop(0, CH * C // 16, zbody, 0, unroll=8)

    def drain(b, nrows=CH):
        pltpu.make_async_copy(
            bufs[b].at[pl.ds(0, nrows * C)],
            out_hbm.at[pl.ds(0, nrows * C)], sems[b]).wait()

    def clear_chunk(k, b):
        for g in range(G):
            old = idx_v[pl.ds(off + k * CH + g * 16, 16)]
            plsc.store_scatter(bufs[b], [rows0 + g * 16 * C + old], zeros)

    def ones_chunk(k, b, ngroups=G):
        for g in range(ngroups):
            cols = idx_v[pl.ds(off + k * CH + g * 16, 16)]
            plsc.store_scatter(bufs[b], [rows0 + g * 16 * C + cols], ones)

    def dma_out(k, b, nrows=CH):
        pltpu.async_copy(
            bufs[b].at[pl.ds(0, nrows * C)],
            out_hbm.at[pl.ds((base + k * CH) * C, nrows * C)],
            sems[b])

    for k in (0, 1):
        ones_chunk(k, k)
        dma_out(k, k)

    def loop_body(j, carry):
        k = 2 + 2 * j
        for b in range(2):
            drain(b)
            clear_chunk(k + b - 2, b)
            ones_chunk(k + b, b)
            dma_out(k + b, b)
        return carry
    lax.fori_loop(0, (FL - 2) // 2, loop_body, 0)

    for k in range(FL, NCHUNK):
        b = k % 2
        drain(b)
        clear_chunk(k - 2, b)

        @pl.when(jnp.logical_not(islast))
        def _():
            ones_chunk(k, b)
            dma_out(k, b)

        if k == FL:
            @pl.when(islast)
            def _():
                ones_chunk(k, b, ngroups=GL)
                dma_out(k, b, nrows=TL)

    for kk in (NCHUNK - 2, NCHUNK - 1):
        b = kk % 2
        if kk < FL:
            drain(b)
        else:
            @pl.when(jnp.logical_not(islast))
            def _():
                drain(b)

            if kk == FL:
                @pl.when(islast)
                def _():
                    drain(b, nrows=TL)


_onehot_sc = pl.kernel(
    _onehot_body,
    out_type=jax.ShapeDtypeStruct((NLOC * C,), jnp.int32),
    mesh=plsc.VectorSubcoreMesh(core_axis_name="c", subcore_axis_name="s"),
    compiler_params=pltpu.CompilerParams(needs_layout_passes=False),
    scratch_types=[
        pltpu.VMEM((IDXA,), jnp.int32),
        pltpu.VMEM((CH * C,), jnp.int32),
        pltpu.VMEM((CH * C,), jnp.int32),
        pltpu.SemaphoreType.DMA,
        pltpu.SemaphoreType.DMA,
    ],
)

_mesh = jax.make_mesh((NDEV,), ("d",),
                      axis_types=(jax.sharding.AxisType.Auto,))


def _shard_fn(w_full):
    dev = lax.axis_index("d")
    w_local = lax.dynamic_slice(w_full, (dev * NLOC,), (NLOC,))
    return _onehot_sc(w_local)


def kernel(x, weight):
    del x
    out = jax.shard_map(
        _shard_fn, mesh=_mesh, in_specs=P(None), out_specs=P("d"))(weight)
    return jax.lax.with_sharding_constraint(
        out.reshape(N, C), jax.NamedSharding(_mesh, P("d", None)))


# speedup vs baseline: 12.1962x; 3.4674x over previous
import jax
import jax.numpy as jnp
from jax import lax
from jax.experimental import pallas as pl
from jax.experimental.pallas import tpu as pltpu
from jax.experimental.pallas import tpu_sc as plsc

N = 100000
C = 128
NC, NS = 2, 16
NW = NC * NS
CH = 448
G = CH // 16

NCHUNK = -(-N // (NW * CH))
R = NCHUNK * CH
RL = N - (NW - 1) * R
FL = RL // CH
TL = RL - FL * CH
GL = TL // 16
IDXA = NW * R - N + R
assert RL > 0 and RL % 16 == 0 and TL % 16 == 0
assert FL >= 2 and FL % 2 == 0 and GL > 0


def _onehot_body(idx_hbm, out_hbm, idx_v, buf0, buf1, sem0, sem1, isem):
    bufs = (buf0, buf1)
    sems = (sem0, sem1)
    wid = lax.axis_index("s") * NC + lax.axis_index("c")
    islast = wid == NW - 1
    base = wid * R
    abase = pl.multiple_of(jnp.minimum(base, N - R), 8)
    off = base - abase

    stage = pltpu.async_copy(
        idx_hbm.at[pl.ds(abase, R)], idx_v.at[pl.ds(0, R)], isem)

    rows0 = lax.iota(jnp.int32, 16) * C
    ones = jnp.ones((16,), jnp.int32)
    zeros = jnp.zeros((16,), jnp.int32)

    def zbody(i, carry):
        buf0[pl.ds(i * 16, 16)] = zeros
        buf1[pl.ds(i * 16, 16)] = zeros
        return carry
    lax.fori_loop(0, CH * C // 16, zbody, 0, unroll=8)
    stage.wait()

    def drain(b, nrows=CH):
        pltpu.make_async_copy(
            bufs[b].at[pl.ds(0, nrows * C)],
            out_hbm.at[pl.ds(0, nrows * C)], sems[b]).wait()

    def clear_chunk(k, b):
        for g in range(G):
            old = idx_v[pl.ds(off + k * CH + g * 16, 16)]
            plsc.store_scatter(bufs[b], [rows0 + g * 16 * C + old], zeros)

    def ones_chunk(k, b, ngroups=G):
        for g in range(ngroups):
            cols = idx_v[pl.ds(off + k * CH + g * 16, 16)]
            plsc.store_scatter(bufs[b], [rows0 + g * 16 * C + cols], ones)

    def dma_out(k, b, nrows=CH):
        pltpu.async_copy(
            bufs[b].at[pl.ds(0, nrows * C)],
            out_hbm.at[pl.ds((base + k * CH) * C, nrows * C)],
            sems[b])

    for k in (0, 1):
        ones_chunk(k, k)
        dma_out(k, k)

    def loop_body(j, carry):
        k = 2 + 2 * j
        for b in range(2):
            drain(b)
            clear_chunk(k + b - 2, b)
            ones_chunk(k + b, b)
            dma_out(k + b, b)
        return carry
    if FL > 2:
        lax.fori_loop(0, (FL - 2) // 2, loop_body, 0)

    for k in range(FL, NCHUNK):
        b = k % 2
        drain(b)
        clear_chunk(k - 2, b)

        @pl.when(jnp.logical_not(islast))
        def _():
            ones_chunk(k, b)
            dma_out(k, b)

        if k == FL:
            @pl.when(islast)
            def _():
                ones_chunk(k, b, ngroups=GL)
                dma_out(k, b, nrows=TL)

    for kk in (NCHUNK - 2, NCHUNK - 1):
        b = kk % 2
        if kk < FL:
            drain(b)
        else:
            @pl.when(jnp.logical_not(islast))
            def _():
                drain(b)

            if kk == FL:
                @pl.when(islast)
                def _():
                    drain(b, nrows=TL)


_onehot_sc = pl.kernel(
    _onehot_body,
    out_type=jax.ShapeDtypeStruct((N * C,), jnp.int32),
    mesh=plsc.VectorSubcoreMesh(core_axis_name="c", subcore_axis_name="s"),
    compiler_params=pltpu.CompilerParams(needs_layout_passes=False),
    scratch_types=[
        pltpu.VMEM((IDXA,), jnp.int32),
        pltpu.VMEM((CH * C,), jnp.int32),
        pltpu.VMEM((CH * C,), jnp.int32),
        pltpu.SemaphoreType.DMA,
        pltpu.SemaphoreType.DMA,
        pltpu.SemaphoreType.DMA,
    ],
)


def kernel(x, weight):
    del x
    return _onehot_sc(weight).reshape(N, C)


# speedup vs baseline: 13.0743x; 1.0720x over previous
import jax
import jax.numpy as jnp
from jax import lax
from jax.experimental import pallas as pl
from jax.experimental.pallas import tpu as pltpu
from jax.experimental.pallas import tpu_sc as plsc

N = 100000
C = 128
NC, NS = 2, 16
NW = NC * NS
CH = 224
G = CH // 16

NCHUNK = -(-N // (NW * CH))
R = NCHUNK * CH
RL = N - (NW - 1) * R
FL = RL // CH
TL = RL - FL * CH
GL = TL // 16
IDXA = NW * R - N + R
assert RL > 0 and RL % 16 == 0 and TL % 16 == 0
assert FL >= 2 and FL % 2 == 0 and GL > 0


def _onehot_body(idx_hbm, out_hbm, idx_v, buf0, buf1, sem0, sem1, isem):
    bufs = (buf0, buf1)
    sems = (sem0, sem1)
    wid = lax.axis_index("s") * NC + lax.axis_index("c")
    islast = wid == NW - 1
    base = wid * R
    abase = pl.multiple_of(jnp.minimum(base, N - R), 8)
    off = base - abase

    stage = pltpu.async_copy(
        idx_hbm.at[pl.ds(abase, R)], idx_v.at[pl.ds(0, R)], isem)

    rows0 = lax.iota(jnp.int32, 16) * C
    ones = jnp.ones((16,), jnp.int32)
    zeros = jnp.zeros((16,), jnp.int32)

    def zbody(i, carry):
        buf0[pl.ds(i * 16, 16)] = zeros
        buf1[pl.ds(i * 16, 16)] = zeros
        return carry
    lax.fori_loop(0, CH * C // 16, zbody, 0, unroll=8)
    stage.wait()

    def drain(b, nrows=CH):
        pltpu.make_async_copy(
            bufs[b].at[pl.ds(0, nrows * C)],
            out_hbm.at[pl.ds(0, nrows * C)], sems[b]).wait()

    def clear_chunk(k, b):
        for g in range(G):
            old = idx_v[pl.ds(off + k * CH + g * 16, 16)]
            plsc.store_scatter(bufs[b], [rows0 + g * 16 * C + old], zeros)

    def ones_chunk(k, b, ngroups=G):
        for g in range(ngroups):
            cols = idx_v[pl.ds(off + k * CH + g * 16, 16)]
            plsc.store_scatter(bufs[b], [rows0 + g * 16 * C + cols], ones)

    def dma_out(k, b, nrows=CH):
        pltpu.async_copy(
            bufs[b].at[pl.ds(0, nrows * C)],
            out_hbm.at[pl.ds((base + k * CH) * C, nrows * C)],
            sems[b])

    for k in (0, 1):
        ones_chunk(k, k)
        dma_out(k, k)

    def loop_body(j, carry):
        k = 2 + 2 * j
        for b in range(2):
            drain(b)
            clear_chunk(k + b - 2, b)
            ones_chunk(k + b, b)
            dma_out(k + b, b)
        return carry
    if FL > 2:
        lax.fori_loop(0, (FL - 2) // 2, loop_body, 0)

    for k in range(FL, NCHUNK):
        b = k % 2
        drain(b)
        clear_chunk(k - 2, b)

        @pl.when(jnp.logical_not(islast))
        def _():
            ones_chunk(k, b)
            dma_out(k, b)

        if k == FL:
            @pl.when(islast)
            def _():
                ones_chunk(k, b, ngroups=GL)
                dma_out(k, b, nrows=TL)

    for kk in (NCHUNK - 2, NCHUNK - 1):
        b = kk % 2
        if kk < FL:
            drain(b)
        else:
            @pl.when(jnp.logical_not(islast))
            def _():
                drain(b)

            if kk == FL:
                @pl.when(islast)
                def _():
                    drain(b, nrows=TL)


_onehot_sc = pl.kernel(
    _onehot_body,
    out_type=jax.ShapeDtypeStruct((N * C,), jnp.int32),
    mesh=plsc.VectorSubcoreMesh(core_axis_name="c", subcore_axis_name="s"),
    compiler_params=pltpu.CompilerParams(needs_layout_passes=False),
    scratch_types=[
        pltpu.VMEM((IDXA,), jnp.int32),
        pltpu.VMEM((CH * C,), jnp.int32),
        pltpu.VMEM((CH * C,), jnp.int32),
        pltpu.SemaphoreType.DMA,
        pltpu.SemaphoreType.DMA,
        pltpu.SemaphoreType.DMA,
    ],
)


def kernel(x, weight):
    del x
    return _onehot_sc(weight).reshape(N, C)
